# baseline (speedup 1.0000x reference)
import functools

import jax
import jax.numpy as jnp
import numpy as np
from jax import lax
from jax.experimental import pallas as pl
from jax.experimental.pallas import tpu as pltpu
from jax.experimental.pallas import tpu_sc as plsc

N = 10000
E = 320000
NC = 2
NS = 16
NW = NC * NS
C = 128
NCHUNK = E // C
KFULL = NCHUNK // NW
KEXTRA = NCHUNK % NW
RPT = 624


def _perm(d):
  p = np.empty(d, np.int32)
  for g in range(d // 32):
    for i in range(16):
      p[32 * g + 2 * i] = 32 * g + i
      p[32 * g + 2 * i + 1] = 32 * g + 16 + i
  return p


_P128 = _perm(128)
_P64 = _perm(64)
_PMAT128 = np.eye(128, dtype=np.float32)[_P128].T


def _make_spmm(d, tc_tiling=False):
  mesh = plsc.VectorSubcoreMesh(core_axis_name="c", subcore_axis_name="s")
  ng = d // 32

  @functools.partial(
      pl.kernel,
      out_type=jax.ShapeDtypeStruct((2 * N, d), jnp.float32),
      mesh=mesh,
      compiler_params=pltpu.CompilerParams(
          needs_layout_passes=False, use_tc_tiling_on_sc=tc_tiling),
      scratch_types=[
          [pltpu.VMEM((2, C), jnp.int32) for _ in range(3)],
          [pltpu.VMEM((C,), jnp.float32) for _ in range(3)],
          [pltpu.VMEM((C,), jnp.int32) for _ in range(2)],
          [pltpu.VMEM((C, d), jnp.bfloat16) for _ in range(2)],
          [pltpu.VMEM((C, d), jnp.float32) for _ in range(2)],
          pltpu.VMEM_SHARED((N, d), jnp.float32),
          [pltpu.SemaphoreType.DMA for _ in range(3)],
          [pltpu.SemaphoreType.DMA for _ in range(2)],
          [pltpu.SemaphoreType.DMA for _ in range(2)],
      ],
  )
  def spmm(mat_hbm, ei_hbm, vals_hbm, out_hbm,
           eiv, valsv, rsc, gb, gf, acc, si, sg, ss):
    cid = lax.axis_index("c")
    sid = lax.axis_index("s")
    wid = cid * NS + sid

    def base_of(cloc):
      return jnp.minimum((wid + NW * cloc) * C, E - C)

    def fire_idx(cloc, j):
      base = base_of(cloc)
      pltpu.async_copy(ei_hbm.at[:, pl.ds(base, C)], eiv[j], si[j])
      pltpu.async_copy(vals_hbm.at[pl.ds(base, C)], valsv[j], si[j])

    def wait_idx(j):
      pltpu.make_async_copy(ei_hbm.at[:, pl.ds(0, C)], eiv[j], si[j]).wait()
      pltpu.make_async_copy(vals_hbm.at[pl.ds(0, C)], valsv[j], si[j]).wait()

    def fire_gather(j, b):
      pltpu.async_copy(mat_hbm.at[eiv[j].at[1]], gb[b], sg[b])

    def wait_gather(j, b):
      pltpu.make_async_copy(mat_hbm.at[eiv[j].at[1]], gb[b], sg[b]).wait()

    def fire_scatter(b):
      pltpu.async_copy(gf[b], acc.at[rsc[b]], ss[b], add=True)

    def wait_scatter(b):
      pltpu.make_async_copy(gf[b], acc.at[rsc[b]], ss[b]).wait()

    def copy_row_to_rsc(j, b):
      for t in range(C // 16):
        rsc[b][pl.ds(t * 16, 16)] = eiv[j][0, pl.ds(t * 16, 16)]

    mask_hi = jnp.full((16,), -65536, jnp.int32)

    def scale(j, b):
      @plsc.parallel_loop(0, C, step=4, unroll=4)
      def body(i2):
        for u in range(4):
          i = i2 + u
          bv = plsc.load_gather(valsv[j], [jnp.full((16,), i, jnp.int32)])
          for t in range(ng):
            xi = plsc.bitcast(gb[b][i, pl.ds(t * 32, 32)], jnp.int32)
            lo = plsc.bitcast(xi << 16, jnp.float32)
            hi = plsc.bitcast(xi & mask_hi, jnp.float32)
            gf[b][i, pl.ds(t * 32, 16)] = lo * bv
            gf[b][i, pl.ds(t * 32 + 16, 16)] = hi * bv

    zero16f = jnp.zeros((16,), jnp.float32)
    zero16i = jnp.zeros((16,), jnp.int32)

    def grow(i, carry):
      for b in range(2):
        for t in range(d // 16):
          gf[b][i, pl.ds(t * 16, 16)] = zero16f
      return carry

    lax.fori_loop(0, C, grow, 0)
    for b in range(2):
      for t in range(C // 16):
        rsc[b][pl.ds(t * 16, 16)] = zero16i

    for k in range(RPT // C):
      pltpu.sync_copy(gf[0], acc.at[pl.ds(sid * RPT + k * C, C)])
    rem = RPT - (RPT // C) * C
    pltpu.sync_copy(gf[0].at[pl.ds(0, rem)],
                    acc.at[pl.ds(sid * RPT + (RPT // C) * C, rem)])

    @pl.when(sid == NS - 1)
    def _():
      pltpu.sync_copy(gf[0].at[pl.ds(0, N - NS * RPT)],
                      acc.at[pl.ds(NS * RPT, N - NS * RPT)])

    plsc.subcore_barrier()

    fire_idx(0, 0)
    fire_idx(1, 1)
    fire_idx(2, 2)
    fire_scatter(0)
    fire_scatter(1)
    wait_idx(0)
    fire_gather(0, 0)

    def phase(k, u):
      cloc = 6 * k + u
      j = u % 3
      jn = (u + 1) % 3
      b = u % 2
      bn = (u + 1) % 2
      wait_gather(j, b)
      wait_idx(jn)
      fire_gather(jn, bn)
      wait_scatter(b)
      copy_row_to_rsc(j, b)
      scale(j, b)
      fire_scatter(b)
      fire_idx(cloc + 3, j)
      return cloc

    def iteration(k, carry):
      for u in range(6):
        phase(k, u)
      return carry

    lax.fori_loop(0, KFULL // 6, iteration, 0)

    wait_gather(0, 0)
    wait_scatter(0)
    wait_scatter(1)

    @pl.when(wid < KEXTRA)
    def _():
      copy_row_to_rsc(0, 0)
      scale(0, 0)
      pltpu.sync_copy(gf[0], acc.at[rsc[0]], add=True)

    wait_idx(1)
    wait_idx(2)

    plsc.subcore_barrier()
    pltpu.sync_copy(acc.at[pl.ds(sid * RPT, RPT)],
                    out_hbm.at[pl.ds(cid * N + sid * RPT, RPT)])

    @pl.when(sid == NS - 1)
    def _():
      pltpu.sync_copy(acc.at[pl.ds(NS * RPT, N - NS * RPT)],
                      out_hbm.at[pl.ds(cid * N + NS * RPT, N - NS * RPT)])

  return spmm


_spmm128 = _make_spmm(128)
_spmm64 = _make_spmm(64)

_BR = 1000
_NB = N // _BR


def _combine_relu_perm(p):
  def body(a_ref, b_ref, m_ref, o_ref):
    h = jnp.maximum(a_ref[...] + b_ref[...], 0.0)
    o_ref[...] = jnp.dot(
        h, m_ref[...], preferred_element_type=jnp.float32
    ).astype(jnp.bfloat16)

  return pl.pallas_call(
      body,
      grid=(_NB,),
      in_specs=[
          pl.BlockSpec((_BR, 128), lambda i: (i, 0)),
          pl.BlockSpec((_BR, 128), lambda i: (_NB + i, 0)),
          pl.BlockSpec((128, 128), lambda i: (0, 0)),
      ],
      out_specs=pl.BlockSpec((_BR, 128), lambda i: (i, 0)),
      out_shape=jax.ShapeDtypeStruct((N, 128), jnp.bfloat16),
  )(p, p, jnp.asarray(_PMAT128))


def _combine_relu_linear(q, wp, biasp):
  def body(a_ref, b_ref, w_ref, bias_ref, o_ref):
    h = jnp.maximum(a_ref[...] + b_ref[...], 0.0)
    o_ref[...] = (
        jnp.dot(h, w_ref[...].T, preferred_element_type=jnp.float32)
        + bias_ref[...]
    ).astype(jnp.bfloat16)

  return pl.pallas_call(
      body,
      grid=(_NB,),
      in_specs=[
          pl.BlockSpec((_BR, 128), lambda i: (i, 0)),
          pl.BlockSpec((_BR, 128), lambda i: (_NB + i, 0)),
          pl.BlockSpec((64, 128), lambda i: (0, 0)),
          pl.BlockSpec((1, 64), lambda i: (0, 0)),
      ],
      out_specs=pl.BlockSpec((_BR, 64), lambda i: (i, 0)),
      out_shape=jax.ShapeDtypeStruct((N, 64), jnp.bfloat16),
  )(q, q, wp, biasp)


_RW = 312


@functools.partial(
    pl.kernel,
    out_type=jax.ShapeDtypeStruct((N, 64), jnp.float32),
    mesh=plsc.VectorSubcoreMesh(core_axis_name="c", subcore_axis_name="s"),
    compiler_params=pltpu.CompilerParams(
        needs_layout_passes=False, use_tc_tiling_on_sc=False),
    scratch_types=[
        pltpu.VMEM((_RW, 64), jnp.float32),
        pltpu.VMEM((_RW, 64), jnp.float32),
    ],
)
def _combine_final(r_hbm, out_hbm, av, bv):
  cid = lax.axis_index("c")
  sid = lax.axis_index("s")
  wid = cid * NS + sid
  base = wid * _RW

  def do_block(nrows, base_):
    pltpu.sync_copy(r_hbm.at[pl.ds(base_, nrows)], av.at[pl.ds(0, nrows)])
    pltpu.sync_copy(r_hbm.at[pl.ds(N + base_, nrows)], bv.at[pl.ds(0, nrows)])

    def add_row(i, carry):
      for t in range(4):
        sl = pl.ds(t * 16, 16)
        av[i, sl] = av[i, sl] + bv[i, sl]
      return carry

    lax.fori_loop(0, nrows, add_row, 0)
    pltpu.sync_copy(av.at[pl.ds(0, nrows)], out_hbm.at[pl.ds(base_, nrows)])

  do_block(_RW, base)

  @pl.when(wid == NW - 1)
  def _():
    do_block(N - NW * _RW, NW * _RW)


def kernel(x, edge_index, adj_vals, W_weight, W_bias):
  xp = x[:, _P128].astype(jnp.bfloat16)
  wp = W_weight[_P64]
  biasp = W_bias[_P64].reshape(1, 64)

  p = _spmm128(xp, edge_index, adj_vals)
  h1 = _combine_relu_perm(p)
  q = _spmm128(h1, edge_index, adj_vals)
  support = _combine_relu_linear(q, wp, biasp)
  r = _spmm64(support, edge_index, adj_vals)
  return _combine_final(r)


# speedup vs baseline: 1.0259x; 1.0259x over previous
import functools

import jax
import jax.numpy as jnp
import numpy as np
from jax import lax
from jax.experimental import pallas as pl
from jax.experimental.pallas import tpu as pltpu
from jax.experimental.pallas import tpu_sc as plsc

N = 10000
E = 320000
NC = 2
NS = 16
NW = NC * NS
C = 128
NCHUNK = E // C
KFULL = NCHUNK // NW
KEXTRA = NCHUNK % NW
RPT = 624


def _perm(d):
  p = np.empty(d, np.int32)
  for g in range(d // 32):
    for i in range(16):
      p[32 * g + 2 * i] = 32 * g + i
      p[32 * g + 2 * i + 1] = 32 * g + 16 + i
  return p


_P128 = _perm(128)
_P64 = _perm(64)
_PMAT128 = np.eye(128, dtype=np.float32)[_P128].T


def _make_spmm(d, tc_tiling=False):
  mesh = plsc.VectorSubcoreMesh(core_axis_name="c", subcore_axis_name="s")
  ng = d // 32

  @functools.partial(
      pl.kernel,
      out_type=jax.ShapeDtypeStruct((2 * N, d), jnp.float32),
      mesh=mesh,
      compiler_params=pltpu.CompilerParams(
          needs_layout_passes=False, use_tc_tiling_on_sc=tc_tiling),
      scratch_types=[
          [pltpu.VMEM((2, C), jnp.int32) for _ in range(3)],
          [pltpu.VMEM((C,), jnp.float32) for _ in range(3)],
          [pltpu.VMEM((C,), jnp.int32) for _ in range(2)],
          [pltpu.VMEM((C, d), jnp.bfloat16) for _ in range(2)],
          [pltpu.VMEM((C, d), jnp.float32) for _ in range(2)],
          pltpu.VMEM_SHARED((N, d), jnp.float32),
          [pltpu.SemaphoreType.DMA for _ in range(3)],
          [pltpu.SemaphoreType.DMA for _ in range(2)],
          [pltpu.SemaphoreType.DMA for _ in range(2)],
      ],
  )
  def spmm(mat_hbm, ei_hbm, vals_hbm, out_hbm,
           eiv, valsv, rsc, gb, gf, acc, si, sg, ss):
    cid = lax.axis_index("c")
    sid = lax.axis_index("s")
    wid = cid * NS + sid

    def base_of(cloc):
      return jnp.minimum((wid + NW * cloc) * C, E - C)

    def fire_idx(cloc, j):
      base = base_of(cloc)
      pltpu.async_copy(ei_hbm.at[:, pl.ds(base, C)], eiv[j], si[j])
      pltpu.async_copy(vals_hbm.at[pl.ds(base, C)], valsv[j], si[j])

    def wait_idx(j):
      pltpu.make_async_copy(ei_hbm.at[:, pl.ds(0, C)], eiv[j], si[j]).wait()
      pltpu.make_async_copy(vals_hbm.at[pl.ds(0, C)], valsv[j], si[j]).wait()

    def fire_gather(j, b):
      pltpu.async_copy(mat_hbm.at[eiv[j].at[1]], gb[b], sg[b])

    def wait_gather(j, b):
      pltpu.make_async_copy(mat_hbm.at[eiv[j].at[1]], gb[b], sg[b]).wait()

    def fire_scatter(b):
      pltpu.async_copy(gf[b], acc.at[rsc[b]], ss[b], add=True)

    def wait_scatter(b):
      pltpu.make_async_copy(gf[b], acc.at[rsc[b]], ss[b]).wait()

    def copy_row_to_rsc(j, b):
      for t in range(C // 16):
        rsc[b][pl.ds(t * 16, 16)] = eiv[j][0, pl.ds(t * 16, 16)]

    mask_hi = jnp.full((16,), -65536, jnp.int32)

    def scale(j, b):
      @plsc.parallel_loop(0, C, step=2, unroll=2)
      def body(i2):
        for u in range(2):
          i = i2 + u
          bv = plsc.load_gather(valsv[j], [jnp.full((16,), i, jnp.int32)])
          for t in range(ng):
            xi = plsc.bitcast(gb[b][i, pl.ds(t * 32, 32)], jnp.int32)
            lo = plsc.bitcast(xi << 16, jnp.float32)
            hi = plsc.bitcast(xi & mask_hi, jnp.float32)
            gf[b][i, pl.ds(t * 32, 16)] = lo * bv
            gf[b][i, pl.ds(t * 32 + 16, 16)] = hi * bv

    zero16f = jnp.zeros((16,), jnp.float32)
    zero16i = jnp.zeros((16,), jnp.int32)

    def grow(i, carry):
      for b in range(2):
        for t in range(d // 16):
          gf[b][i, pl.ds(t * 16, 16)] = zero16f
      return carry

    lax.fori_loop(0, C, grow, 0)
    for b in range(2):
      for t in range(C // 16):
        rsc[b][pl.ds(t * 16, 16)] = zero16i

    for k in range(RPT // C):
      pltpu.sync_copy(gf[0], acc.at[pl.ds(sid * RPT + k * C, C)])
    rem = RPT - (RPT // C) * C
    pltpu.sync_copy(gf[0].at[pl.ds(0, rem)],
                    acc.at[pl.ds(sid * RPT + (RPT // C) * C, rem)])

    @pl.when(sid == NS - 1)
    def _():
      pltpu.sync_copy(gf[0].at[pl.ds(0, N - NS * RPT)],
                      acc.at[pl.ds(NS * RPT, N - NS * RPT)])

    plsc.subcore_barrier()

    fire_idx(0, 0)
    fire_idx(1, 1)
    fire_idx(2, 2)
    fire_scatter(0)
    fire_scatter(1)
    wait_idx(0)
    fire_gather(0, 0)

    def phase(k, u):
      cloc = 6 * k + u
      j = u % 3
      jn = (u + 1) % 3
      b = u % 2
      bn = (u + 1) % 2
      wait_gather(j, b)
      wait_idx(jn)
      fire_gather(jn, bn)
      wait_scatter(b)
      copy_row_to_rsc(j, b)
      scale(j, b)
      fire_scatter(b)
      fire_idx(cloc + 3, j)
      return cloc

    def iteration(k, carry):
      for u in range(6):
        phase(k, u)
      return carry

    lax.fori_loop(0, KFULL // 6, iteration, 0)

    wait_gather(0, 0)
    wait_scatter(0)
    wait_scatter(1)

    @pl.when(wid < KEXTRA)
    def _():
      copy_row_to_rsc(0, 0)
      scale(0, 0)
      pltpu.sync_copy(gf[0], acc.at[rsc[0]], add=True)

    wait_idx(1)
    wait_idx(2)

    plsc.subcore_barrier()
    pltpu.sync_copy(acc.at[pl.ds(sid * RPT, RPT)],
                    out_hbm.at[pl.ds(cid * N + sid * RPT, RPT)])

    @pl.when(sid == NS - 1)
    def _():
      pltpu.sync_copy(acc.at[pl.ds(NS * RPT, N - NS * RPT)],
                      out_hbm.at[pl.ds(cid * N + NS * RPT, N - NS * RPT)])

  return spmm


_spmm128 = _make_spmm(128)
_spmm64 = _make_spmm(64)

_BR = 1000
_NB = N // _BR


def _combine_relu_perm(p):
  def body(a_ref, b_ref, m_ref, o_ref):
    h = jnp.maximum(a_ref[...] + b_ref[...], 0.0)
    o_ref[...] = jnp.dot(
        h, m_ref[...], preferred_element_type=jnp.float32
    ).astype(jnp.bfloat16)

  return pl.pallas_call(
      body,
      grid=(_NB,),
      in_specs=[
          pl.BlockSpec((_BR, 128), lambda i: (i, 0)),
          pl.BlockSpec((_BR, 128), lambda i: (_NB + i, 0)),
          pl.BlockSpec((128, 128), lambda i: (0, 0)),
      ],
      out_specs=pl.BlockSpec((_BR, 128), lambda i: (i, 0)),
      out_shape=jax.ShapeDtypeStruct((N, 128), jnp.bfloat16),
  )(p, p, jnp.asarray(_PMAT128))


def _combine_relu_linear(q, wp, biasp):
  def body(a_ref, b_ref, w_ref, bias_ref, o_ref):
    h = jnp.maximum(a_ref[...] + b_ref[...], 0.0)
    o_ref[...] = (
        jnp.dot(h, w_ref[...].T, preferred_element_type=jnp.float32)
        + bias_ref[...]
    ).astype(jnp.bfloat16)

  return pl.pallas_call(
      body,
      grid=(_NB,),
      in_specs=[
          pl.BlockSpec((_BR, 128), lambda i: (i, 0)),
          pl.BlockSpec((_BR, 128), lambda i: (_NB + i, 0)),
          pl.BlockSpec((64, 128), lambda i: (0, 0)),
          pl.BlockSpec((1, 64), lambda i: (0, 0)),
      ],
      out_specs=pl.BlockSpec((_BR, 64), lambda i: (i, 0)),
      out_shape=jax.ShapeDtypeStruct((N, 64), jnp.bfloat16),
  )(q, q, wp, biasp)


_RW = 312


@functools.partial(
    pl.kernel,
    out_type=jax.ShapeDtypeStruct((N, 64), jnp.float32),
    mesh=plsc.VectorSubcoreMesh(core_axis_name="c", subcore_axis_name="s"),
    compiler_params=pltpu.CompilerParams(
        needs_layout_passes=False, use_tc_tiling_on_sc=False),
    scratch_types=[
        pltpu.VMEM((_RW, 64), jnp.float32),
        pltpu.VMEM((_RW, 64), jnp.float32),
    ],
)
def _combine_final(r_hbm, out_hbm, av, bv):
  cid = lax.axis_index("c")
  sid = lax.axis_index("s")
  wid = cid * NS + sid
  base = wid * _RW

  def do_block(nrows, base_):
    pltpu.sync_copy(r_hbm.at[pl.ds(base_, nrows)], av.at[pl.ds(0, nrows)])
    pltpu.sync_copy(r_hbm.at[pl.ds(N + base_, nrows)], bv.at[pl.ds(0, nrows)])

    def add_row(i, carry):
      for t in range(4):
        sl = pl.ds(t * 16, 16)
        av[i, sl] = av[i, sl] + bv[i, sl]
      return carry

    lax.fori_loop(0, nrows, add_row, 0)
    pltpu.sync_copy(av.at[pl.ds(0, nrows)], out_hbm.at[pl.ds(base_, nrows)])

  do_block(_RW, base)

  @pl.when(wid == NW - 1)
  def _():
    do_block(N - NW * _RW, NW * _RW)


def kernel(x, edge_index, adj_vals, W_weight, W_bias):
  xp = x[:, _P128].astype(jnp.bfloat16)
  wp = W_weight[_P64]
  biasp = W_bias[_P64].reshape(1, 64)

  p = _spmm128(xp, edge_index, adj_vals)
  h1 = _combine_relu_perm(p)
  q = _spmm128(h1, edge_index, adj_vals)
  support = _combine_relu_linear(q, wp, biasp)
  r = _spmm64(support, edge_index, adj_vals)
  return _combine_final(r)
